# baseline (speedup 1.0000x reference)
import jax
import jax.numpy as jnp
from jax.experimental import pallas as pl
from jax.experimental.pallas import tpu as pltpu

_N = 10000
_C = 128
_ROWS = 50
_A = 200
_NBLK = _N // _A


def _body(x_ref, o_ref):
    o_ref[...] = jnp.zeros(o_ref.shape, o_ref.dtype)
    x = x_ref[...]
    o_ref[:, 0, :] = x[:, :_C]
    o_ref[:, 25, :] = x[:, _C:]


def kernel(atom_embeddings):
    return pl.pallas_call(
        _body,
        grid=(_NBLK,),
        in_specs=[pl.BlockSpec((_A, 2 * _C), lambda i: (i, 0))],
        out_specs=pl.BlockSpec((_A, _ROWS, _C), lambda i: (i, 0, 0)),
        out_shape=jax.ShapeDtypeStruct((_N, _ROWS, _C), atom_embeddings.dtype),
        compiler_params=pltpu.CompilerParams(
            dimension_semantics=("parallel",),
        ),
    )(atom_embeddings)


# speedup vs baseline: 1.0018x; 1.0018x over previous
import jax
import jax.numpy as jnp
from jax.experimental import pallas as pl
from jax.experimental.pallas import tpu as pltpu

_N = 10000
_C = 128
_ROWS = 50
_A = 1000
_NBLK = _N // _A


def _body(x_hbm, o_hbm, zbuf, xv, isem, zsem, dsem):
    ic = pltpu.make_async_copy(x_hbm, xv, isem)
    ic.start()
    zbuf[...] = jnp.zeros(zbuf.shape, zbuf.dtype)
    zc = []
    for b in range(_NBLK):
        a0 = b * _A
        for j, r0 in enumerate((1, 26)):
            c = pltpu.make_async_copy(
                zbuf,
                o_hbm.at[pl.ds(a0, _A), pl.ds(r0, 24), :],
                zsem.at[b, j],
            )
            c.start()
            zc.append(c)
    ic.wait()
    dc = []
    for b in range(_NBLK):
        a0 = b * _A
        for j, r0 in enumerate((0, 25)):
            c = pltpu.make_async_copy(
                xv.at[pl.ds(a0, _A), pl.ds(j, 1), :],
                o_hbm.at[pl.ds(a0, _A), pl.ds(r0, 1), :],
                dsem.at[b, j],
            )
            c.start()
            dc.append(c)
    for c in zc:
        c.wait()
    for c in dc:
        c.wait()


def kernel(atom_embeddings):
    x3 = atom_embeddings.reshape(_N, 2, _C)
    return pl.pallas_call(
        _body,
        in_specs=[pl.BlockSpec(memory_space=pltpu.MemorySpace.HBM)],
        out_specs=pl.BlockSpec(memory_space=pltpu.MemorySpace.HBM),
        out_shape=jax.ShapeDtypeStruct((_N, _ROWS, _C), atom_embeddings.dtype),
        scratch_shapes=[
            pltpu.VMEM((_A, 24, _C), jnp.float32),
            pltpu.VMEM((_N, 2, _C), jnp.float32),
            pltpu.SemaphoreType.DMA,
            pltpu.SemaphoreType.DMA((_NBLK, 2)),
            pltpu.SemaphoreType.DMA((_NBLK, 2)),
        ],
    )(x3)
